# baseline (speedup 1.0000x reference)
import functools

import jax
import jax.numpy as jnp
from jax import lax
from jax.experimental import pallas as pl
from jax.experimental.pallas import tpu as pltpu
from jax.experimental.pallas import tpu_sc as plsc

_NC = 2
_NS = 16
_G = 80
_ZR = 32
_NB = 3
_NI = 8
_ZR2 = 8


def _sc_accumulate(src, recv, ds_out):
    e_total = src.shape[0]
    n, d = ds_out.shape
    nw = _NC * _NS
    e_per_tile = e_total // nw
    n_chunks = e_per_tile // _G
    n_pad = ((n + 127) // 128) * 128
    rows_per_tile = n_pad // _NS
    n_zero = rows_per_tile // _ZR2

    mesh = plsc.VectorSubcoreMesh(core_axis_name="c", subcore_axis_name="s")

    @functools.partial(
        pl.kernel,
        out_type=(
            jax.ShapeDtypeStruct((_NC, n_pad, d), jnp.float32),
            jax.ShapeDtypeStruct((nw * n_pad,), jnp.float32),
        ),
        mesh=mesh,
        compiler_params=pltpu.CompilerParams(needs_layout_passes=False),
        scratch_types=[
            pltpu.VMEM((_NI, _G), jnp.int32),
            pltpu.VMEM((_NI, _G), jnp.int32),
            pltpu.VMEM((_NB, _G, d), jnp.float32),
            pltpu.VMEM((_ZR2, d), jnp.float32),
            pltpu.VMEM((n_pad,), jnp.float32),
            pltpu.VMEM_SHARED((n_pad, d), jnp.float32),
            pltpu.SemaphoreType.DMA((_NI,)),
            pltpu.SemaphoreType.DMA((_NB,)),
            pltpu.SemaphoreType.DMA((_NB,)),
        ],
    )
    def accumulate(src_hbm, recv_hbm, dsout_hbm, pmsg_hbm, cnt_hbm,
                   sidx, ridx, rows, zb, lhist, msg_sh, isem, gsem, ssem):
        c = lax.axis_index("c")
        s = lax.axis_index("s")
        wid = s * _NC + c

        z16 = jnp.zeros((16,), jnp.float32)
        o16 = jnp.ones((16,), jnp.float32)

        def fill_z(i, carry):
            for j in range(d // 16):
                zb[i, pl.ds(j * 16, 16)] = z16
            return carry

        lax.fori_loop(0, _ZR2, fill_z, 0)

        def fill_h(i, carry):
            lhist[pl.ds(i * 16, 16)] = z16
            return carry

        lax.fori_loop(0, n_pad // 16, fill_h, 0)

        row0 = s * rows_per_tile

        def zero_step(k, carry):
            pltpu.sync_copy(zb, msg_sh.at[pl.ds(row0 + k * _ZR2, _ZR2)])
            return carry

        lax.fori_loop(0, n_zero, zero_step, 0)
        plsc.subcore_barrier()

        ebase = wid * e_per_tile

        def start_idx(chunk, slot):
            off = ebase + chunk * _G
            pltpu.async_copy(src_hbm.at[pl.ds(off, _G)], sidx.at[slot],
                             isem.at[slot])
            pltpu.async_copy(recv_hbm.at[pl.ds(off, _G)], ridx.at[slot],
                             isem.at[slot])

        def wait_idx(chunk, slot):
            off = ebase + chunk * _G
            pltpu.make_async_copy(src_hbm.at[pl.ds(off, _G)], sidx.at[slot],
                                  isem.at[slot]).wait()
            pltpu.make_async_copy(recv_hbm.at[pl.ds(off, _G)], ridx.at[slot],
                                  isem.at[slot]).wait()

        def wait_scatter(slot, islot):
            pltpu.make_async_copy(rows.at[slot], msg_sh.at[ridx.at[islot]],
                                  ssem.at[slot]).wait()

        start_idx(0, 0)
        start_idx(1, 1)
        wait_idx(0, 0)
        pltpu.async_copy(dsout_hbm.at[sidx.at[0]], rows.at[0], gsem.at[0])

        def edge_step(i, carry):
            b = lax.rem(i, _NB)
            bi = lax.rem(i, _NI)
            pltpu.make_async_copy(dsout_hbm.at[sidx.at[bi]], rows.at[b],
                                  gsem.at[b]).wait()
            pltpu.async_copy(rows.at[b], msg_sh.at[ridx.at[bi]], ssem.at[b],
                             add=True)

            @pl.when(i >= 2)
            def _():
                wait_scatter(lax.rem(i + 1, _NB), lax.rem(i - 2, _NI))

            @pl.when(i + 2 < n_chunks)
            def _():
                start_idx(i + 2, lax.rem(i + 2, _NI))

            @pl.when(i + 1 < n_chunks)
            def _():
                bn = lax.rem(i + 1, _NB)
                bni = lax.rem(i + 1, _NI)
                wait_idx(i + 1, bni)
                pltpu.async_copy(dsout_hbm.at[sidx.at[bni]], rows.at[bn],
                                 gsem.at[bn])

            for j in range(_G // 16):
                idx16 = ridx[bi, pl.ds(j * 16, 16)]
                plsc.addupdate_scatter(lhist, [idx16], o16)
            return carry

        lax.fori_loop(0, n_chunks, edge_step, 0)

        wait_scatter((n_chunks - 2) % _NB, (n_chunks - 2) % _NI)
        wait_scatter((n_chunks - 1) % _NB, (n_chunks - 1) % _NI)
        plsc.subcore_barrier()

        pltpu.sync_copy(msg_sh.at[pl.ds(row0, rows_per_tile)],
                        pmsg_hbm.at[c, pl.ds(row0, rows_per_tile)])
        pltpu.sync_copy(lhist, cnt_hbm.at[pl.ds(wid * n_pad, n_pad)])

    pmsg, cnt = accumulate(src, recv, ds_out)
    return pmsg, cnt.reshape(nw, n_pad)


def _combine_body(di_ref, pm_ref, pc_ref, o_ref):
    m = di_ref[...] + pm_ref[0] + pm_ref[1]
    den = 1.0 + jnp.sum(pc_ref[...], axis=1, keepdims=True)
    o_ref[...] = m / den


def _combine(ds_in, pmsg, pcnt_t):
    n, d = ds_in.shape
    br = 1000
    nw = pcnt_t.shape[1]
    return pl.pallas_call(
        _combine_body,
        grid=(n // br,),
        in_specs=[
            pl.BlockSpec((br, d), lambda i: (i, 0)),
            pl.BlockSpec((_NC, br, d), lambda i: (0, i, 0)),
            pl.BlockSpec((br, nw), lambda i: (i, 0)),
        ],
        out_specs=pl.BlockSpec((br, d), lambda i: (i, 0)),
        out_shape=jax.ShapeDtypeStruct((n, d), jnp.float32),
    )(ds_in, pmsg, pcnt_t)


def kernel(ds_in, ds_out, layer_edge_index):
    lei = layer_edge_index.astype(jnp.int32)
    pmsg, cnt = _sc_accumulate(lei[0], lei[1], ds_out)
    return _combine(ds_in, pmsg, cnt.T)


# speedup vs baseline: 1.1210x; 1.1210x over previous
import functools

import jax
import jax.numpy as jnp
from jax import lax
from jax.experimental import pallas as pl
from jax.experimental.pallas import tpu as pltpu
from jax.experimental.pallas import tpu_sc as plsc

_NC = 2
_NS = 16
_G = 40
_ZR = 32
_NB = 6
_NI = 8
_ZR2 = 8


def _sc_accumulate(src, recv, ds_out):
    e_total = src.shape[0]
    n, d = ds_out.shape
    nw = _NC * _NS
    e_per_tile = e_total // nw
    n_chunks = e_per_tile // _G
    n_pad = ((n + 127) // 128) * 128
    rows_per_tile = n_pad // _NS
    n_zero = rows_per_tile // _ZR2

    mesh = plsc.VectorSubcoreMesh(core_axis_name="c", subcore_axis_name="s")

    @functools.partial(
        pl.kernel,
        out_type=(
            jax.ShapeDtypeStruct((_NC, n_pad, d), jnp.float32),
            jax.ShapeDtypeStruct((nw * n_pad,), jnp.float32),
        ),
        mesh=mesh,
        compiler_params=pltpu.CompilerParams(needs_layout_passes=False),
        scratch_types=[
            pltpu.VMEM((_NI, _G), jnp.int32),
            pltpu.VMEM((_NI, _G), jnp.int32),
            pltpu.VMEM((_NB, _G, d), jnp.float32),
            pltpu.VMEM((_ZR2, d), jnp.float32),
            pltpu.VMEM((n_pad,), jnp.float32),
            pltpu.VMEM_SHARED((n_pad, d), jnp.float32),
            pltpu.SemaphoreType.DMA((_NI,)),
            pltpu.SemaphoreType.DMA((_NB,)),
            pltpu.SemaphoreType.DMA((_NB,)),
            pltpu.SemaphoreType.DMA,
        ],
    )
    def accumulate(src_hbm, recv_hbm, dsout_hbm, pmsg_hbm, cnt_hbm,
                   sidx, ridx, rows, zb, lhist, msg_sh, isem, gsem, ssem,
                   zsem):
        c = lax.axis_index("c")
        s = lax.axis_index("s")
        wid = s * _NC + c

        z16 = jnp.zeros((16,), jnp.float32)
        o16 = jnp.ones((16,), jnp.float32)

        def fill_z(i, carry):
            for j in range(d // 16):
                zb[i, pl.ds(j * 16, 16)] = z16
            return carry

        lax.fori_loop(0, _ZR2, fill_z, 0)

        def fill_h(i, carry):
            lhist[pl.ds(i * 16, 16)] = z16
            return carry

        lax.fori_loop(0, n_pad // 16, fill_h, 0)

        row0 = s * rows_per_tile

        def zero_start(k, carry):
            pltpu.async_copy(zb, msg_sh.at[pl.ds(row0 + k * _ZR2, _ZR2)],
                             zsem)
            return carry

        lax.fori_loop(0, n_zero, zero_start, 0)

        def zero_drain(k, carry):
            pltpu.make_async_copy(zb, msg_sh.at[pl.ds(row0, _ZR2)],
                                  zsem).wait()
            return carry

        lax.fori_loop(0, n_zero, zero_drain, 0)
        plsc.subcore_barrier()

        ebase = wid * e_per_tile

        def start_idx(chunk, slot):
            off = ebase + chunk * _G
            pltpu.async_copy(src_hbm.at[pl.ds(off, _G)], sidx.at[slot],
                             isem.at[slot])
            pltpu.async_copy(recv_hbm.at[pl.ds(off, _G)], ridx.at[slot],
                             isem.at[slot])

        def wait_idx(chunk, slot):
            off = ebase + chunk * _G
            pltpu.make_async_copy(src_hbm.at[pl.ds(off, _G)], sidx.at[slot],
                                  isem.at[slot]).wait()
            pltpu.make_async_copy(recv_hbm.at[pl.ds(off, _G)], ridx.at[slot],
                                  isem.at[slot]).wait()

        def wait_scatter(slot, islot):
            pltpu.make_async_copy(rows.at[slot], msg_sh.at[ridx.at[islot]],
                                  ssem.at[slot]).wait()

        start_idx(0, 0)
        start_idx(1, 1)
        start_idx(2, 2)
        start_idx(3, 3)
        wait_idx(0, 0)
        pltpu.async_copy(dsout_hbm.at[sidx.at[0]], rows.at[0], gsem.at[0])
        wait_idx(1, 1)
        pltpu.async_copy(dsout_hbm.at[sidx.at[1]], rows.at[1], gsem.at[1])

        def edge_step(i, carry):
            b = lax.rem(i, _NB)
            bi = lax.rem(i, _NI)
            pltpu.make_async_copy(dsout_hbm.at[sidx.at[bi]], rows.at[b],
                                  gsem.at[b]).wait()
            pltpu.async_copy(rows.at[b], msg_sh.at[ridx.at[bi]], ssem.at[b],
                             add=True)

            @pl.when(i >= 2)
            def _():
                wait_scatter(lax.rem(i - 2, _NB), lax.rem(i - 2, _NI))

            @pl.when(i + 4 < n_chunks)
            def _():
                start_idx(i + 4, lax.rem(i + 4, _NI))

            @pl.when(i + 2 < n_chunks)
            def _():
                bn = lax.rem(i + 2, _NB)
                bni = lax.rem(i + 2, _NI)
                wait_idx(i + 2, bni)
                pltpu.async_copy(dsout_hbm.at[sidx.at[bni]], rows.at[bn],
                                 gsem.at[bn])

            for j in range(_G // 16):
                idx16 = ridx[bi, pl.ds(j * 16, 16)]
                plsc.addupdate_scatter(lhist, [idx16], o16)
            return carry

        lax.fori_loop(0, n_chunks, edge_step, 0)

        wait_scatter((n_chunks - 2) % _NB, (n_chunks - 2) % _NI)
        wait_scatter((n_chunks - 1) % _NB, (n_chunks - 1) % _NI)
        plsc.subcore_barrier()

        pltpu.sync_copy(msg_sh.at[pl.ds(row0, rows_per_tile)],
                        pmsg_hbm.at[c, pl.ds(row0, rows_per_tile)])
        pltpu.sync_copy(lhist, cnt_hbm.at[pl.ds(wid * n_pad, n_pad)])

    pmsg, cnt = accumulate(src, recv, ds_out)
    return pmsg, cnt.reshape(nw, n_pad)


def _combine_body(di_ref, pm_ref, pc_ref, o_ref):
    m = di_ref[...] + pm_ref[0] + pm_ref[1]
    den = 1.0 + jnp.sum(pc_ref[...], axis=1, keepdims=True)
    o_ref[...] = m / den


def _combine(ds_in, pmsg, pcnt_t):
    n, d = ds_in.shape
    br = 1000
    nw = pcnt_t.shape[1]
    return pl.pallas_call(
        _combine_body,
        grid=(n // br,),
        in_specs=[
            pl.BlockSpec((br, d), lambda i: (i, 0)),
            pl.BlockSpec((_NC, br, d), lambda i: (0, i, 0)),
            pl.BlockSpec((br, nw), lambda i: (i, 0)),
        ],
        out_specs=pl.BlockSpec((br, d), lambda i: (i, 0)),
        out_shape=jax.ShapeDtypeStruct((n, d), jnp.float32),
    )(ds_in, pmsg, pcnt_t)


def kernel(ds_in, ds_out, layer_edge_index):
    lei = layer_edge_index.astype(jnp.int32)
    pmsg, cnt = _sc_accumulate(lei[0], lei[1], ds_out)
    return _combine(ds_in, pmsg, cnt.T)


# speedup vs baseline: 1.1224x; 1.0012x over previous
import functools

import jax
import jax.numpy as jnp
from jax import lax
from jax.experimental import pallas as pl
from jax.experimental.pallas import tpu as pltpu
from jax.experimental.pallas import tpu_sc as plsc

_NC = 2
_NS = 16
_G = 40
_ZR = 32
_NB = 6
_NI = 8
_ZR2 = 8


def _sc_accumulate(src, recv, ds_out):
    e_total = src.shape[0]
    n, d = ds_out.shape
    nw = _NC * _NS
    e_per_tile = e_total // nw
    n_chunks = e_per_tile // _G
    n_pad = ((n + 127) // 128) * 128
    rows_per_tile = n_pad // _NS
    n_zero = rows_per_tile // _ZR2

    mesh = plsc.VectorSubcoreMesh(core_axis_name="c", subcore_axis_name="s")

    @functools.partial(
        pl.kernel,
        out_type=(
            jax.ShapeDtypeStruct((_NC, n_pad, d), jnp.float32),
            jax.ShapeDtypeStruct((nw * n_pad,), jnp.float32),
        ),
        mesh=mesh,
        compiler_params=pltpu.CompilerParams(needs_layout_passes=False),
        scratch_types=[
            pltpu.VMEM((_NI, _G), jnp.int32),
            pltpu.VMEM((_NI, _G), jnp.int32),
            pltpu.VMEM((_NB, _G, d), jnp.float32),
            pltpu.VMEM((_ZR2, d), jnp.float32),
            pltpu.VMEM((n_pad,), jnp.float32),
            pltpu.VMEM_SHARED((n_pad, d), jnp.float32),
            pltpu.SemaphoreType.DMA((_NI,)),
            pltpu.SemaphoreType.DMA((_NB,)),
            pltpu.SemaphoreType.DMA((_NB,)),
            pltpu.SemaphoreType.DMA,
        ],
    )
    def accumulate(src_hbm, recv_hbm, dsout_hbm, pmsg_hbm, cnt_hbm,
                   sidx, ridx, rows, zb, lhist, msg_sh, isem, gsem, ssem,
                   zsem):
        c = lax.axis_index("c")
        s = lax.axis_index("s")
        wid = s * _NC + c

        z16 = jnp.zeros((16,), jnp.float32)
        o16 = jnp.ones((16,), jnp.float32)

        def fill_z(i, carry):
            for j in range(d // 16):
                zb[i, pl.ds(j * 16, 16)] = z16
            return carry

        lax.fori_loop(0, _ZR2, fill_z, 0)

        def fill_h(i, carry):
            lhist[pl.ds(i * 16, 16)] = z16
            return carry

        lax.fori_loop(0, n_pad // 16, fill_h, 0)

        row0 = s * rows_per_tile

        def zero_start(k, carry):
            pltpu.async_copy(zb, msg_sh.at[pl.ds(row0 + k * _ZR2, _ZR2)],
                             zsem)
            return carry

        lax.fori_loop(0, n_zero, zero_start, 0)

        def zero_drain(k, carry):
            pltpu.make_async_copy(zb, msg_sh.at[pl.ds(row0, _ZR2)],
                                  zsem).wait()
            return carry

        lax.fori_loop(0, n_zero, zero_drain, 0)
        plsc.subcore_barrier()

        ebase = wid * e_per_tile

        def start_idx(chunk, slot):
            off = ebase + chunk * _G
            pltpu.async_copy(src_hbm.at[pl.ds(off, _G)], sidx.at[slot],
                             isem.at[slot])
            pltpu.async_copy(recv_hbm.at[pl.ds(off, _G)], ridx.at[slot],
                             isem.at[slot])

        def wait_idx(chunk, slot):
            off = ebase + chunk * _G
            pltpu.make_async_copy(src_hbm.at[pl.ds(off, _G)], sidx.at[slot],
                                  isem.at[slot]).wait()
            pltpu.make_async_copy(recv_hbm.at[pl.ds(off, _G)], ridx.at[slot],
                                  isem.at[slot]).wait()

        def wait_scatter(slot, islot):
            pltpu.make_async_copy(rows.at[slot], msg_sh.at[ridx.at[islot]],
                                  ssem.at[slot]).wait()

        start_idx(0, 0)
        start_idx(1, 1)
        start_idx(2, 2)
        start_idx(3, 3)
        wait_idx(0, 0)
        pltpu.async_copy(dsout_hbm.at[sidx.at[0]], rows.at[0], gsem.at[0])
        wait_idx(1, 1)
        pltpu.async_copy(dsout_hbm.at[sidx.at[1]], rows.at[1], gsem.at[1])

        def edge_step(i, carry):
            b = lax.rem(i, _NB)
            bi = lax.rem(i, _NI)
            pltpu.make_async_copy(dsout_hbm.at[sidx.at[bi]], rows.at[b],
                                  gsem.at[b]).wait()
            pltpu.async_copy(rows.at[b], msg_sh.at[ridx.at[bi]], ssem.at[b],
                             add=True)

            @pl.when(i >= 2)
            def _():
                wait_scatter(lax.rem(i - 2, _NB), lax.rem(i - 2, _NI))

            @pl.when(i + 4 < n_chunks)
            def _():
                start_idx(i + 4, lax.rem(i + 4, _NI))

            @pl.when(i + 2 < n_chunks)
            def _():
                bn = lax.rem(i + 2, _NB)
                bni = lax.rem(i + 2, _NI)
                wait_idx(i + 2, bni)
                pltpu.async_copy(dsout_hbm.at[sidx.at[bni]], rows.at[bn],
                                 gsem.at[bn])

            for j in range(_G // 16):
                idx16 = ridx[bi, pl.ds(j * 16, 16)]
                plsc.addupdate_scatter(lhist, [idx16], o16)
            if _G % 16:
                tail = _G % 16
                idx16 = ridx[bi, pl.ds(_G - 16, 16)]
                m = lax.iota(jnp.int32, 16) >= (16 - tail)
                plsc.addupdate_scatter(lhist, [idx16], o16, mask=m)
            return carry

        lax.fori_loop(0, n_chunks, edge_step, 0)

        wait_scatter((n_chunks - 2) % _NB, (n_chunks - 2) % _NI)
        wait_scatter((n_chunks - 1) % _NB, (n_chunks - 1) % _NI)
        plsc.subcore_barrier()

        pltpu.sync_copy(msg_sh.at[pl.ds(row0, rows_per_tile)],
                        pmsg_hbm.at[c, pl.ds(row0, rows_per_tile)])
        pltpu.sync_copy(lhist, cnt_hbm.at[pl.ds(wid * n_pad, n_pad)])

    pmsg, cnt = accumulate(src, recv, ds_out)
    return pmsg, cnt.reshape(nw, n_pad)


def _combine_body(di_ref, pm_ref, pc_ref, o_ref):
    m = di_ref[...] + pm_ref[0] + pm_ref[1]
    den = 1.0 + jnp.sum(pc_ref[...], axis=1, keepdims=True)
    o_ref[...] = m / den


def _combine(ds_in, pmsg, pcnt_t):
    n, d = ds_in.shape
    br = 1000
    nw = pcnt_t.shape[1]
    return pl.pallas_call(
        _combine_body,
        grid=(n // br,),
        in_specs=[
            pl.BlockSpec((br, d), lambda i: (i, 0)),
            pl.BlockSpec((_NC, br, d), lambda i: (0, i, 0)),
            pl.BlockSpec((br, nw), lambda i: (i, 0)),
        ],
        out_specs=pl.BlockSpec((br, d), lambda i: (i, 0)),
        out_shape=jax.ShapeDtypeStruct((n, d), jnp.float32),
    )(ds_in, pmsg, pcnt_t)


def kernel(ds_in, ds_out, layer_edge_index):
    lei = layer_edge_index.astype(jnp.int32)
    pmsg, cnt = _sc_accumulate(lei[0], lei[1], ds_out)
    return _combine(ds_in, pmsg, cnt.T)


# speedup vs baseline: 1.4560x; 1.2972x over previous
import functools

import jax
import jax.numpy as jnp
from jax import lax
from jax.experimental import pallas as pl
from jax.experimental.pallas import tpu as pltpu
from jax.experimental.pallas import tpu_sc as plsc

_NC = 2
_NS = 16
_G = 40
_ZR = 32
_NB = 7
_NI = 8
_ZR2 = 8


def _sc_accumulate(src, recv, ds_out):
    e_total = src.shape[0]
    n, d = ds_out.shape
    nw = _NC * _NS
    e_per_tile = e_total // nw
    n_chunks = e_per_tile // _G
    n_pad = ((n + 127) // 128) * 128
    rows_per_tile = n_pad // _NS
    n_zero = rows_per_tile // _ZR2

    mesh = plsc.VectorSubcoreMesh(core_axis_name="c", subcore_axis_name="s")

    @functools.partial(
        pl.kernel,
        out_type=(
            jax.ShapeDtypeStruct((_NC, n_pad, d), jnp.float32),
            jax.ShapeDtypeStruct((nw * n_pad,), jnp.float32),
        ),
        mesh=mesh,
        compiler_params=pltpu.CompilerParams(needs_layout_passes=False),
        scratch_types=[
            pltpu.VMEM((_NI, _G), jnp.int32),
            pltpu.VMEM((_NI, _G), jnp.int32),
            pltpu.VMEM((_NB, _G, d), jnp.float32),
            pltpu.VMEM((_ZR2, d), jnp.float32),
            pltpu.VMEM((n_pad,), jnp.float32),
            pltpu.VMEM_SHARED((n_pad, d), jnp.float32),
            pltpu.SemaphoreType.DMA((_NI,)),
            pltpu.SemaphoreType.DMA((_NB,)),
            pltpu.SemaphoreType.DMA((_NB,)),
            pltpu.SemaphoreType.DMA,
        ],
    )
    def accumulate(src_hbm, recv_hbm, dsout_hbm, pmsg_hbm, cnt_hbm,
                   sidx, ridx, rows, zb, lhist, msg_sh, isem, gsem, ssem,
                   zsem):
        c = lax.axis_index("c")
        s = lax.axis_index("s")
        wid = s * _NC + c

        z16 = jnp.zeros((16,), jnp.float32)
        o16 = jnp.ones((16,), jnp.float32)

        def fill_z(i, carry):
            for j in range(d // 16):
                zb[i, pl.ds(j * 16, 16)] = z16
            return carry

        lax.fori_loop(0, _ZR2, fill_z, 0)

        def fill_h(i, carry):
            lhist[pl.ds(i * 16, 16)] = z16
            return carry

        lax.fori_loop(0, n_pad // 16, fill_h, 0)

        row0 = s * rows_per_tile

        def zero_start(k, carry):
            pltpu.async_copy(zb, msg_sh.at[pl.ds(row0 + k * _ZR2, _ZR2)],
                             zsem)
            return carry

        lax.fori_loop(0, n_zero, zero_start, 0)

        ebase = wid * e_per_tile

        def start_idx(chunk, slot):
            off = ebase + chunk * _G
            pltpu.async_copy(src_hbm.at[pl.ds(off, _G)], sidx.at[slot],
                             isem.at[slot])
            pltpu.async_copy(recv_hbm.at[pl.ds(off, _G)], ridx.at[slot],
                             isem.at[slot])

        def wait_idx(chunk, slot):
            off = ebase + chunk * _G
            pltpu.make_async_copy(src_hbm.at[pl.ds(off, _G)], sidx.at[slot],
                                  isem.at[slot]).wait()
            pltpu.make_async_copy(recv_hbm.at[pl.ds(off, _G)], ridx.at[slot],
                                  isem.at[slot]).wait()

        def wait_scatter(slot, islot):
            pltpu.make_async_copy(rows.at[slot], msg_sh.at[ridx.at[islot]],
                                  ssem.at[slot]).wait()

        for k in range(6):
            start_idx(k, k)
        for k in range(4):
            wait_idx(k, k)
            pltpu.async_copy(dsout_hbm.at[sidx.at[k]], rows.at[k],
                             gsem.at[k])

        def zero_drain(k, carry):
            pltpu.make_async_copy(zb, msg_sh.at[pl.ds(row0, _ZR2)],
                                  zsem).wait()
            return carry

        lax.fori_loop(0, n_zero, zero_drain, 0)
        plsc.subcore_barrier()

        def edge_step(i, carry):
            b = lax.rem(i, _NB)
            bi = lax.rem(i, _NI)
            pltpu.make_async_copy(dsout_hbm.at[sidx.at[bi]], rows.at[b],
                                  gsem.at[b]).wait()
            pltpu.async_copy(rows.at[b], msg_sh.at[ridx.at[bi]], ssem.at[b],
                             add=True)

            @pl.when(i >= 2)
            def _():
                wait_scatter(lax.rem(i - 2, _NB), lax.rem(i - 2, _NI))

            @pl.when(i + 6 < n_chunks)
            def _():
                start_idx(i + 6, lax.rem(i + 6, _NI))

            @pl.when(i + 4 < n_chunks)
            def _():
                bn = lax.rem(i + 4, _NB)
                bni = lax.rem(i + 4, _NI)
                wait_idx(i + 4, bni)
                pltpu.async_copy(dsout_hbm.at[sidx.at[bni]], rows.at[bn],
                                 gsem.at[bn])

            for j in range(_G // 16):
                idx16 = ridx[bi, pl.ds(j * 16, 16)]
                plsc.addupdate_scatter(lhist, [idx16], o16)
            if _G % 16:
                tail = _G % 16
                idx16 = ridx[bi, pl.ds(_G - 16, 16)]
                m = lax.iota(jnp.int32, 16) >= (16 - tail)
                plsc.addupdate_scatter(lhist, [idx16], o16, mask=m)
            return carry

        lax.fori_loop(0, n_chunks, edge_step, 0)

        wait_scatter((n_chunks - 2) % _NB, (n_chunks - 2) % _NI)
        wait_scatter((n_chunks - 1) % _NB, (n_chunks - 1) % _NI)
        plsc.subcore_barrier()

        pltpu.sync_copy(msg_sh.at[pl.ds(row0, rows_per_tile)],
                        pmsg_hbm.at[c, pl.ds(row0, rows_per_tile)])
        pltpu.sync_copy(lhist, cnt_hbm.at[pl.ds(wid * n_pad, n_pad)])

    pmsg, cnt = accumulate(src, recv, ds_out)
    return pmsg, cnt.reshape(nw, n_pad)


def _combine_body(di_ref, pm_ref, pc_ref, o_ref):
    m = di_ref[...] + pm_ref[0] + pm_ref[1]
    den = 1.0 + jnp.sum(pc_ref[...], axis=1, keepdims=True)
    o_ref[...] = m / den


def _combine(ds_in, pmsg, pcnt_t):
    n, d = ds_in.shape
    br = 1000
    nw = pcnt_t.shape[1]
    return pl.pallas_call(
        _combine_body,
        grid=(n // br,),
        in_specs=[
            pl.BlockSpec((br, d), lambda i: (i, 0)),
            pl.BlockSpec((_NC, br, d), lambda i: (0, i, 0)),
            pl.BlockSpec((br, nw), lambda i: (i, 0)),
        ],
        out_specs=pl.BlockSpec((br, d), lambda i: (i, 0)),
        out_shape=jax.ShapeDtypeStruct((n, d), jnp.float32),
    )(ds_in, pmsg, pcnt_t)


def kernel(ds_in, ds_out, layer_edge_index):
    lei = layer_edge_index.astype(jnp.int32)
    pmsg, cnt = _sc_accumulate(lei[0], lei[1], ds_out)
    return _combine(ds_in, pmsg, cnt.T)
